# baseline (speedup 1.0000x reference)
import functools

import jax
import jax.numpy as jnp
from jax import lax
from jax.experimental import pallas as pl
from jax.experimental.pallas import tpu as pltpu
from jax.experimental.pallas import tpu_sc as plsc

B = 4
N = 50000
F = 128
K = 25000
N2 = 51200
W = 2048
NW = N2 // W
KOUT = 26624
KW = KOUT // W
RADIX = 2048
NEG_INF = float("-inf")

_mesh = plsc.VectorSubcoreMesh(core_axis_name="c", subcore_axis_name="s")


def _lane():
    return lax.iota(jnp.int32, 16)


def _rank16(d, vt_ref):
    lane = _lane()
    kk = d * 16 + lane
    kks, lns = plsc.sort_key_val(kk, lane)
    ds_ = lax.shift_right_logical(kks, jnp.full((16,), 4, jnp.int32))
    vt_ref[...] = ds_
    prev = plsc.load_gather(vt_ref, [jnp.maximum(lane - 1, 0)])
    nxt = plsc.load_gather(vt_ref, [jnp.minimum(lane + 1, 15)])
    is_start = (lane == 0) | (ds_ != prev)
    is_last = (lane == 15) | (ds_ != nxt)
    startpos = plsc.cummax(jnp.where(is_start, lane, 0))
    runpos = lane - startpos
    return ds_, lns, runpos, is_last


def _digit(kd, shift):
    sv = jnp.full((16,), shift, jnp.int32)
    return jnp.bitwise_and(lax.shift_right_logical(kd, sv), RADIX - 1)


def _encode(s):
    s = jnp.where(s == 0.0, 0.0, s)
    u = lax.bitcast_convert_type(s, jnp.int32)
    return jnp.where(u < 0, u, jnp.bitwise_and(~u, 0x7FFFFFFF))


def _decode(kd):
    k = ~kd
    u = jnp.where(k < 0, jnp.bitwise_xor(k, jnp.int32(-2147483648)), ~k)
    return lax.bitcast_convert_type(u, jnp.float32)


@functools.partial(
    pl.kernel,
    out_type=[
        jax.ShapeDtypeStruct((B, KOUT), jnp.int32),
        jax.ShapeDtypeStruct((B, KOUT), jnp.float32),
    ],
    scratch_types=[
        pltpu.VMEM_SHARED((2, 2, N2), jnp.int32),
        pltpu.VMEM_SHARED((2, 2, N2), jnp.int32),
        pltpu.VMEM((W,), jnp.float32),
        pltpu.VMEM((W,), jnp.int32),
        pltpu.VMEM((W,), jnp.int32),
        pltpu.VMEM((16, 128), jnp.int32),
        pltpu.VMEM((16, 128), jnp.int32),
        pltpu.VMEM((16, 128), jnp.int32),
        pltpu.VMEM((RADIX,), jnp.int32),
        pltpu.VMEM((16,), jnp.int32),
        pltpu.VMEM((W,), jnp.float32),
        pltpu.SemaphoreType.DMA,
    ],
    compiler_params=pltpu.CompilerParams(needs_layout_passes=False,
                                         use_tc_tiling_on_sc=False),
    mesh=_mesh,
)
def _sort_kernel(score_hbm, sidx_hbm, sscore_hbm, sk, sv, swin, kwin, vwin,
                 kout, vout, pos, off, vt16, oscore, sem):
    c = lax.axis_index("c")
    s = lax.axis_index("s")
    row = c * 2 + s

    @pl.when(s < 2)
    def _():
        lane = _lane()

        for p in range(3):
            shift = 11 * p
            src = (p + 1) % 2
            dst = p % 2

            def _zero(i, _):
                off[pl.ds(i * 16, 16)] = jnp.zeros((16,), jnp.int32)
                return 0
            lax.fori_loop(0, RADIX // 16, _zero, 0)

            def _hist_win(w, _):
                if p == 0:
                    pltpu.sync_copy(score_hbm.at[row, pl.ds(w * W, W)], swin)
                else:
                    pltpu.sync_copy(sk.at[src, s, pl.ds(w * W, W)], kwin)

                def _hist_vreg(j, _):
                    if p == 0:
                        kd = _encode(swin[pl.ds(j * 16, 16)])
                    else:
                        kd = kwin[pl.ds(j * 16, 16)]
                    d = _digit(kd, shift)
                    ds_, _lns, runpos, is_last = _rank16(d, vt16)
                    g = plsc.load_gather(off, [ds_])
                    plsc.store_scatter(off, [ds_], g + runpos + 1,
                                       mask=is_last)
                    return 0
                lax.fori_loop(0, W // 16, _hist_vreg, 0)
                return 0
            lax.fori_loop(0, NW, _hist_win, 0)

            def _scan(i, carry):
                h = off[pl.ds(i * 16, 16)]
                cs = plsc.cumsum(h)
                off[pl.ds(i * 16, 16)] = cs - h + jnp.full((16,), 1, jnp.int32) * carry
                return carry + jnp.sum(h, axis=0)
            lax.fori_loop(0, RADIX // 16, _scan, jnp.int32(0))

            def _scat_win(w, _):
                if p == 0:
                    pltpu.sync_copy(score_hbm.at[row, pl.ds(w * W, W)], swin)
                else:
                    pltpu.sync_copy(sk.at[src, s, pl.ds(w * W, W)], kwin)
                    pltpu.sync_copy(sv.at[src, s, pl.ds(w * W, W)], vwin)

                def _scat_vreg(j, _):
                    if p == 0:
                        kd = _encode(swin[pl.ds(j * 16, 16)])
                        v = w * W + j * 16 + lane
                    else:
                        kd = kwin[pl.ds(j * 16, 16)]
                        v = vwin[pl.ds(j * 16, 16)]
                    d = _digit(kd, shift)
                    ds_, lns, runpos, is_last = _rank16(d, vt16)
                    base = plsc.load_gather(off, [ds_])
                    p_s = base + runpos
                    plsc.store_scatter(off, [ds_], p_s + 1, mask=is_last)
                    vt16[...] = kd
                    k_s = plsc.load_gather(vt16, [lns])
                    kout[j // 8, pl.ds((j % 8) * 16, 16)] = k_s
                    vt16[...] = v
                    v_s = plsc.load_gather(vt16, [lns])
                    vout[j // 8, pl.ds((j % 8) * 16, 16)] = v_s
                    pos[j // 8, pl.ds((j % 8) * 16, 16)] = p_s
                    return 0
                lax.fori_loop(0, W // 16, _scat_vreg, 0)

                handles = []
                for i in range(16):
                    handles.append(pltpu.async_copy(
                        kout.at[i], sk.at[dst, s].at[pos.at[i]], sem))
                    handles.append(pltpu.async_copy(
                        vout.at[i], sv.at[dst, s].at[pos.at[i]], sem))
                for h in handles:
                    h.wait()
                return 0
            lax.fori_loop(0, NW, _scat_win, 0)

        pltpu.sync_copy(sv.at[0, s, pl.ds(0, KOUT)], sidx_hbm.at[row])

        def _emit_win(w, _):
            pltpu.sync_copy(sk.at[0, s, pl.ds(w * W, W)], kwin)

            def _emit_vreg(j, _):
                oscore[pl.ds(j * 16, 16)] = _decode(kwin[pl.ds(j * 16, 16)])
                return 0
            lax.fori_loop(0, W // 16, _emit_vreg, 0)
            pltpu.sync_copy(oscore, sscore_hbm.at[row, pl.ds(w * W, W)])
            return 0
        lax.fori_loop(0, KW, _emit_win, 0)


CH = 128
PW = 3200
NCH = PW // CH


@functools.partial(
    pl.kernel,
    out_type=[
        jax.ShapeDtypeStruct((B, 8 * PW, F), jnp.float32),
        jax.ShapeDtypeStruct((B, 8 * PW, 16), jnp.float32),
    ],
    scratch_types=[
        pltpu.VMEM((CH,), jnp.int32),
        pltpu.VMEM((CH,), jnp.float32),
        pltpu.VMEM((CH, F), jnp.float32),
        pltpu.VMEM((CH, 16), jnp.float32),
        pltpu.SemaphoreType.DMA,
    ],
    compiler_params=pltpu.CompilerParams(needs_layout_passes=False,
                                         use_tc_tiling_on_sc=False),
    mesh=_mesh,
)
def _gather_kernel(xf_hbm, pp_hbm, sidx_hbm, sscore_hbm, xo_hbm, po_hbm,
                   idxg, sv, xrows, prows, sem):
    c = lax.axis_index("c")
    s = lax.axis_index("s")
    wid = s * 2 + c
    b = wid // 8
    base = (wid % 8) * PW

    def _chunk(t, _):
        o = base + t * CH
        pltpu.sync_copy(sidx_hbm.at[b, pl.ds(o, CH)], idxg)
        for g in range(CH // 16):
            idxg[pl.ds(g * 16, 16)] = idxg[pl.ds(g * 16, 16)] + b * N
        pltpu.sync_copy(sscore_hbm.at[b, pl.ds(o, CH)], sv)
        pltpu.async_copy(xf_hbm.at[idxg], xrows, sem).wait()

        def _scale(r, _):
            bc = plsc.load_gather(sv, [jnp.full((16,), 1, jnp.int32) * r])
            for g in range(F // 16):
                xrows[r, pl.ds(g * 16, 16)] = xrows[r, pl.ds(g * 16, 16)] * bc
            return 0
        lax.fori_loop(0, CH, _scale, 0)
        pltpu.sync_copy(xrows, xo_hbm.at[b, pl.ds(o, CH)])

        pltpu.async_copy(pp_hbm.at[idxg], prows, sem).wait()
        pltpu.sync_copy(prows, po_hbm.at[b, pl.ds(o, CH)])
        return 0
    lax.fori_loop(0, NCH, _chunk, 0)


def kernel(x, posi, weight):
    b, n, f = x.shape
    xf = x.reshape(b * n, f)
    score = jnp.tanh((xf * weight).sum(-1)
                     / jnp.sqrt(jnp.sum(weight * weight, -1))).reshape(b, n)
    score_p = jnp.pad(score, ((0, 0), (0, N2 - n)), constant_values=NEG_INF)
    sidx, sscore = _sort_kernel(score_p)
    pp = jnp.pad(posi, ((0, 0), (0, 0), (0, 13))).reshape(b * n, 16)
    xo, po = _gather_kernel(xf, pp, sidx, sscore)
    k = n // 2
    return (xo[:, :k], po[:, :k, :3])


# speedup vs baseline: 2.2423x; 2.2423x over previous
import functools

import jax
import jax.numpy as jnp
from jax import lax
from jax.experimental import pallas as pl
from jax.experimental.pallas import tpu as pltpu
from jax.experimental.pallas import tpu_sc as plsc

B = 4
N = 50000
F = 128
K = 25000
N2 = 51200
CHUNK = N2 // 8
W = 1280
NW = CHUNK // W
KOUT = 26624
KE = KOUT // 8
RADIX = 2048
NEG_INF = float("-inf")

_mesh = plsc.VectorSubcoreMesh(core_axis_name="c", subcore_axis_name="s")


def _lane():
    return lax.iota(jnp.int32, 16)


def _rank16(d, vt_ref):
    lane = _lane()
    kk = d * 16 + lane
    kks, lns = plsc.sort_key_val(kk, lane)
    ds_ = lax.shift_right_logical(kks, jnp.full((16,), 4, jnp.int32))
    vt_ref[...] = ds_
    prev = plsc.load_gather(vt_ref, [jnp.maximum(lane - 1, 0)])
    nxt = plsc.load_gather(vt_ref, [jnp.minimum(lane + 1, 15)])
    is_start = (lane == 0) | (ds_ != prev)
    is_last = (lane == 15) | (ds_ != nxt)
    startpos = plsc.cummax(jnp.where(is_start, lane, 0))
    runpos = lane - startpos
    return ds_, lns, runpos, is_last


def _digit(kd, shift):
    sv = jnp.full((16,), shift, jnp.int32)
    return jnp.bitwise_and(lax.shift_right_logical(kd, sv), RADIX - 1)


def _encode(s):
    s = jnp.where(s == 0.0, 0.0, s)
    u = lax.bitcast_convert_type(s, jnp.int32)
    return jnp.where(u < 0, u, jnp.bitwise_and(~u, 0x7FFFFFFF))


def _decode(kd):
    k = ~kd
    u = jnp.where(k < 0, jnp.bitwise_xor(k, jnp.int32(-2147483648)), ~k)
    return lax.bitcast_convert_type(u, jnp.float32)


@functools.partial(
    pl.kernel,
    out_type=[
        jax.ShapeDtypeStruct((B, KOUT), jnp.int32),
        jax.ShapeDtypeStruct((B, KOUT), jnp.float32),
    ],
    scratch_types=[
        pltpu.VMEM_SHARED((2, 2, N2), jnp.int32),
        pltpu.VMEM_SHARED((2, 2, N2), jnp.int32),
        pltpu.VMEM_SHARED((16, RADIX), jnp.int32),
        pltpu.VMEM((W,), jnp.float32),
        pltpu.VMEM((W,), jnp.int32),
        pltpu.VMEM((W,), jnp.int32),
        pltpu.VMEM((10, 128), jnp.int32),
        pltpu.VMEM((10, 128), jnp.int32),
        pltpu.VMEM((10, 128), jnp.int32),
        pltpu.VMEM((RADIX,), jnp.int32),
        pltpu.VMEM((8, RADIX), jnp.int32),
        pltpu.VMEM((16,), jnp.int32),
        pltpu.VMEM((KE,), jnp.int32),
        pltpu.VMEM((KE,), jnp.float32),
        pltpu.SemaphoreType.DMA,
    ],
    compiler_params=pltpu.CompilerParams(needs_layout_passes=False,
                                         use_tc_tiling_on_sc=False),
    mesh=_mesh,
)
def _sort_kernel(score_hbm, sidx_hbm, sscore_hbm, sk, sv, hists, swin, kwin,
                 vwin, kout, vout, pos, off, hall, vt16, okey, oscore, sem):
    c = lax.axis_index("c")
    s = lax.axis_index("s")
    r = s // 8
    row = c * 2 + r
    w8 = s % 8
    cbase = w8 * CHUNK
    lane = _lane()

    for p in range(3):
        shift = 11 * p
        src = (p + 1) % 2
        dst = p % 2

        def _zero(i, _):
            off[pl.ds(i * 16, 16)] = jnp.zeros((16,), jnp.int32)
            return 0
        lax.fori_loop(0, RADIX // 16, _zero, 0)

        def _hist_win(w, _):
            if p == 0:
                pltpu.sync_copy(
                    score_hbm.at[row, pl.ds(cbase + w * W, W)], swin)
            else:
                pltpu.sync_copy(sk.at[src, r, pl.ds(cbase + w * W, W)], kwin)

            def _hist_vreg(j, _):
                if p == 0:
                    kd = _encode(swin[pl.ds(j * 16, 16)])
                else:
                    kd = kwin[pl.ds(j * 16, 16)]
                d = _digit(kd, shift)
                ds_, _lns, runpos, is_last = _rank16(d, vt16)
                g = plsc.load_gather(off, [ds_])
                plsc.store_scatter(off, [ds_], g + runpos + 1, mask=is_last)
                return 0
            lax.fori_loop(0, W // 16, _hist_vreg, 0)
            return 0
        lax.fori_loop(0, NW, _hist_win, 0)

        pltpu.sync_copy(off, hists.at[s])
        plsc.subcore_barrier()
        for w2 in range(8):
            pltpu.sync_copy(hists.at[r * 8 + w2], hall.at[w2])

        def _scan(i, carry):
            tot = jnp.zeros((16,), jnp.int32)
            part = jnp.zeros((16,), jnp.int32)
            for w2 in range(8):
                hv = hall[w2, pl.ds(i * 16, 16)]
                tot = tot + hv
                part = jnp.where(w2 < w8, part + hv, part)
            cs = plsc.cumsum(tot)
            off[pl.ds(i * 16, 16)] = (cs - tot) + part \
                + jnp.full((16,), 1, jnp.int32) * carry
            return carry + jnp.sum(tot, axis=0)
        lax.fori_loop(0, RADIX // 16, _scan, jnp.int32(0))

        def _scat_win(w, _):
            if p == 0:
                pltpu.sync_copy(
                    score_hbm.at[row, pl.ds(cbase + w * W, W)], swin)
            else:
                pltpu.sync_copy(sk.at[src, r, pl.ds(cbase + w * W, W)], kwin)
                pltpu.sync_copy(sv.at[src, r, pl.ds(cbase + w * W, W)], vwin)

            def _scat_vreg(j, _):
                if p == 0:
                    kd = _encode(swin[pl.ds(j * 16, 16)])
                    v = cbase + w * W + j * 16 + lane
                else:
                    kd = kwin[pl.ds(j * 16, 16)]
                    v = vwin[pl.ds(j * 16, 16)]
                d = _digit(kd, shift)
                ds_, lns, runpos, is_last = _rank16(d, vt16)
                base = plsc.load_gather(off, [ds_])
                p_s = base + runpos
                plsc.store_scatter(off, [ds_], p_s + 1, mask=is_last)
                vt16[...] = kd
                k_s = plsc.load_gather(vt16, [lns])
                kout[j // 8, pl.ds((j % 8) * 16, 16)] = k_s
                vt16[...] = v
                v_s = plsc.load_gather(vt16, [lns])
                vout[j // 8, pl.ds((j % 8) * 16, 16)] = v_s
                pos[j // 8, pl.ds((j % 8) * 16, 16)] = p_s
                return 0
            lax.fori_loop(0, W // 16, _scat_vreg, 0)

            handles = []
            for i in range(10):
                handles.append(pltpu.async_copy(
                    kout.at[i], sk.at[dst, r].at[pos.at[i]], sem))
                handles.append(pltpu.async_copy(
                    vout.at[i], sv.at[dst, r].at[pos.at[i]], sem))
            for h in handles:
                h.wait()
            return 0
        lax.fori_loop(0, NW, _scat_win, 0)

        plsc.subcore_barrier()

    ebase = w8 * KE
    pltpu.sync_copy(sv.at[0, r, pl.ds(ebase, KE)],
                    sidx_hbm.at[row, pl.ds(ebase, KE)])
    pltpu.sync_copy(sk.at[0, r, pl.ds(ebase, KE)], okey)

    def _emit_vreg(j, _):
        oscore[pl.ds(j * 16, 16)] = _decode(okey[pl.ds(j * 16, 16)])
        return 0
    lax.fori_loop(0, KE // 16, _emit_vreg, 0)
    pltpu.sync_copy(oscore, sscore_hbm.at[row, pl.ds(ebase, KE)])


CH = 128
PW = 3200
NCH = PW // CH


@functools.partial(
    pl.kernel,
    out_type=[
        jax.ShapeDtypeStruct((B, 8 * PW, F), jnp.float32),
        jax.ShapeDtypeStruct((B, 8 * PW, 16), jnp.float32),
    ],
    scratch_types=[
        pltpu.VMEM((CH,), jnp.int32),
        pltpu.VMEM((CH,), jnp.float32),
        pltpu.VMEM((CH, F), jnp.float32),
        pltpu.VMEM((CH, 16), jnp.float32),
        pltpu.SemaphoreType.DMA,
    ],
    compiler_params=pltpu.CompilerParams(needs_layout_passes=False,
                                         use_tc_tiling_on_sc=False),
    mesh=_mesh,
)
def _gather_kernel(xf_hbm, pp_hbm, sidx_hbm, sscore_hbm, xo_hbm, po_hbm,
                   idxg, sv, xrows, prows, sem):
    c = lax.axis_index("c")
    s = lax.axis_index("s")
    wid = s * 2 + c
    b = wid // 8
    base = (wid % 8) * PW

    def _chunk(t, _):
        o = base + t * CH
        pltpu.sync_copy(sidx_hbm.at[b, pl.ds(o, CH)], idxg)
        for g in range(CH // 16):
            idxg[pl.ds(g * 16, 16)] = idxg[pl.ds(g * 16, 16)] + b * N
        pltpu.sync_copy(sscore_hbm.at[b, pl.ds(o, CH)], sv)
        pltpu.async_copy(xf_hbm.at[idxg], xrows, sem).wait()

        def _scale(r, _):
            bc = plsc.load_gather(sv, [jnp.full((16,), 1, jnp.int32) * r])
            for g in range(F // 16):
                xrows[r, pl.ds(g * 16, 16)] = xrows[r, pl.ds(g * 16, 16)] * bc
            return 0
        lax.fori_loop(0, CH, _scale, 0)
        pltpu.sync_copy(xrows, xo_hbm.at[b, pl.ds(o, CH)])

        pltpu.async_copy(pp_hbm.at[idxg], prows, sem).wait()
        pltpu.sync_copy(prows, po_hbm.at[b, pl.ds(o, CH)])
        return 0
    lax.fori_loop(0, NCH, _chunk, 0)


def kernel(x, posi, weight):
    b, n, f = x.shape
    xf = x.reshape(b * n, f)
    score = jnp.tanh((xf * weight).sum(-1)
                     / jnp.sqrt(jnp.sum(weight * weight, -1))).reshape(b, n)
    score_p = jnp.pad(score, ((0, 0), (0, N2 - n)), constant_values=NEG_INF)
    sidx, sscore = _sort_kernel(score_p)
    pp = jnp.pad(posi, ((0, 0), (0, 0), (0, 13))).reshape(b * n, 16)
    xo, po = _gather_kernel(xf, pp, sidx, sscore)
    k = n // 2
    return (xo[:, :k], po[:, :k, :3])


# speedup vs baseline: 2.4878x; 1.1095x over previous
import functools

import jax
import jax.numpy as jnp
from jax import lax
from jax.experimental import pallas as pl
from jax.experimental.pallas import tpu as pltpu
from jax.experimental.pallas import tpu_sc as plsc

B = 4
N = 50000
F = 128
K = 25000
N2 = 51200
CHUNK = N2 // 8
W = 1280
NW = CHUNK // W
KOUT = 26624
KE = KOUT // 8
RADIX = 2048
NEG_INF = float("-inf")

_mesh = plsc.VectorSubcoreMesh(core_axis_name="c", subcore_axis_name="s")


def _lane():
    return lax.iota(jnp.int32, 16)


def _rank16(d, vt_ref):
    lane = _lane()
    kk = d * 16 + lane
    kks, lns = plsc.sort_key_val(kk, lane)
    ds_ = lax.shift_right_logical(kks, jnp.full((16,), 4, jnp.int32))
    vt_ref[...] = ds_
    prev = plsc.load_gather(vt_ref, [jnp.maximum(lane - 1, 0)])
    nxt = plsc.load_gather(vt_ref, [jnp.minimum(lane + 1, 15)])
    is_start = (lane == 0) | (ds_ != prev)
    is_last = (lane == 15) | (ds_ != nxt)
    startpos = plsc.cummax(jnp.where(is_start, lane, 0))
    runpos = lane - startpos
    return ds_, lns, runpos, is_last


def _digit(kd, shift):
    sv = jnp.full((16,), shift, jnp.int32)
    return jnp.bitwise_and(lax.shift_right_logical(kd, sv), RADIX - 1)


def _encode(s):
    s = jnp.where(s == 0.0, 0.0, s)
    u = lax.bitcast_convert_type(s, jnp.int32)
    return jnp.where(u < 0, u, jnp.bitwise_and(~u, 0x7FFFFFFF))


def _decode(kd):
    k = ~kd
    u = jnp.where(k < 0, jnp.bitwise_xor(k, jnp.int32(-2147483648)), ~k)
    return lax.bitcast_convert_type(u, jnp.float32)


@functools.partial(
    pl.kernel,
    out_type=[
        jax.ShapeDtypeStruct((B, KOUT), jnp.int32),
        jax.ShapeDtypeStruct((B, KOUT), jnp.float32),
    ],
    scratch_types=[
        pltpu.VMEM_SHARED((2, 2, N2), jnp.int32),
        pltpu.VMEM_SHARED((2, 2, N2), jnp.int32),
        pltpu.VMEM_SHARED((16, RADIX), jnp.int32),
        pltpu.VMEM((W,), jnp.float32),
        pltpu.VMEM((W,), jnp.int32),
        pltpu.VMEM((W,), jnp.int32),
        pltpu.VMEM((10, 128), jnp.int32),
        pltpu.VMEM((10, 128), jnp.int32),
        pltpu.VMEM((10, 128), jnp.int32),
        pltpu.VMEM((RADIX,), jnp.int32),
        pltpu.VMEM((8, RADIX), jnp.int32),
        pltpu.VMEM((16,), jnp.int32),
        pltpu.VMEM((KE,), jnp.int32),
        pltpu.VMEM((KE,), jnp.float32),
        pltpu.SemaphoreType.DMA,
    ],
    compiler_params=pltpu.CompilerParams(needs_layout_passes=False,
                                         use_tc_tiling_on_sc=False),
    mesh=_mesh,
)
def _sort_kernel(score_hbm, sidx_hbm, sscore_hbm, sk, sv, hists, swin, kwin,
                 vwin, kout, vout, pos, off, hall, vt16, okey, oscore, sem):
    c = lax.axis_index("c")
    s = lax.axis_index("s")
    r = s // 8
    row = c * 2 + r
    w8 = s % 8
    cbase = w8 * CHUNK
    lane = _lane()

    for p in range(3):
        shift = 11 * p
        src = (p + 1) % 2
        dst = p % 2

        def _zero(i, _):
            off[pl.ds(i * 16, 16)] = jnp.zeros((16,), jnp.int32)
            return 0
        lax.fori_loop(0, RADIX // 16, _zero, 0)

        def _hist_win(w, _):
            if p == 0:
                pltpu.sync_copy(
                    score_hbm.at[row, pl.ds(cbase + w * W, W)], swin)
            else:
                pltpu.sync_copy(sk.at[src, r, pl.ds(cbase + w * W, W)], kwin)

            def _hist_vreg(j, _):
                if p == 0:
                    kd = _encode(swin[pl.ds(j * 16, 16)])
                else:
                    kd = kwin[pl.ds(j * 16, 16)]
                d = _digit(kd, shift)
                ds_, _lns, runpos, is_last = _rank16(d, vt16)
                g = plsc.load_gather(off, [ds_])
                plsc.store_scatter(off, [ds_], g + runpos + 1, mask=is_last)
                return 0
            lax.fori_loop(0, W // 16, _hist_vreg, 0)
            return 0
        lax.fori_loop(0, NW, _hist_win, 0)

        pltpu.sync_copy(off, hists.at[s])
        plsc.subcore_barrier()
        for w2 in range(8):
            pltpu.sync_copy(hists.at[r * 8 + w2], hall.at[w2])

        def _scan(i, carry):
            tot = jnp.zeros((16,), jnp.int32)
            part = jnp.zeros((16,), jnp.int32)
            for w2 in range(8):
                hv = hall[w2, pl.ds(i * 16, 16)]
                tot = tot + hv
                part = jnp.where(w2 < w8, part + hv, part)
            cs = plsc.cumsum(tot)
            off[pl.ds(i * 16, 16)] = (cs - tot) + part \
                + jnp.full((16,), 1, jnp.int32) * carry
            return carry + jnp.sum(tot, axis=0)
        lax.fori_loop(0, RADIX // 16, _scan, jnp.int32(0))

        def _scat_win(w, _):
            if p == 0:
                pltpu.sync_copy(
                    score_hbm.at[row, pl.ds(cbase + w * W, W)], swin)
            else:
                pltpu.sync_copy(sk.at[src, r, pl.ds(cbase + w * W, W)], kwin)
                pltpu.sync_copy(sv.at[src, r, pl.ds(cbase + w * W, W)], vwin)

            def _scat_vreg(j, _):
                if p == 0:
                    kd = _encode(swin[pl.ds(j * 16, 16)])
                    v = cbase + w * W + j * 16 + lane
                else:
                    kd = kwin[pl.ds(j * 16, 16)]
                    v = vwin[pl.ds(j * 16, 16)]
                d = _digit(kd, shift)
                ds_, lns, runpos, is_last = _rank16(d, vt16)
                base = plsc.load_gather(off, [ds_])
                p_s = base + runpos
                plsc.store_scatter(off, [ds_], p_s + 1, mask=is_last)
                vt16[...] = kd
                k_s = plsc.load_gather(vt16, [lns])
                kout[j // 8, pl.ds((j % 8) * 16, 16)] = k_s
                vt16[...] = v
                v_s = plsc.load_gather(vt16, [lns])
                vout[j // 8, pl.ds((j % 8) * 16, 16)] = v_s
                pos[j // 8, pl.ds((j % 8) * 16, 16)] = p_s
                return 0
            lax.fori_loop(0, W // 16, _scat_vreg, 0)

            handles = []
            for i in range(10):
                handles.append(pltpu.async_copy(
                    kout.at[i], sk.at[dst, r].at[pos.at[i]], sem))
                handles.append(pltpu.async_copy(
                    vout.at[i], sv.at[dst, r].at[pos.at[i]], sem))
            for h in handles:
                h.wait()
            return 0
        lax.fori_loop(0, NW, _scat_win, 0)

        plsc.subcore_barrier()

    ebase = w8 * KE
    pltpu.sync_copy(sv.at[0, r, pl.ds(ebase, KE)],
                    sidx_hbm.at[row, pl.ds(ebase, KE)])
    pltpu.sync_copy(sk.at[0, r, pl.ds(ebase, KE)], okey)

    def _emit_vreg(j, _):
        oscore[pl.ds(j * 16, 16)] = _decode(okey[pl.ds(j * 16, 16)])
        return 0
    lax.fori_loop(0, KE // 16, _emit_vreg, 0)
    pltpu.sync_copy(oscore, sscore_hbm.at[row, pl.ds(ebase, KE)])


CH = 128
PW = 3200
NCH = PW // CH


@functools.partial(
    pl.kernel,
    out_type=[
        jax.ShapeDtypeStruct((B, 8 * PW, F), jnp.float32),
    ],
    scratch_types=[
        pltpu.VMEM((CH,), jnp.int32),
        pltpu.VMEM((CH,), jnp.float32),
        pltpu.VMEM((CH, F), jnp.float32),
        pltpu.SemaphoreType.DMA,
    ],
    compiler_params=pltpu.CompilerParams(needs_layout_passes=False),
    mesh=_mesh,
)
def _gather_kernel(xf_hbm, sidx_hbm, sscore_hbm, xo_hbm, idxg, sv, xrows,
                   sem):
    c = lax.axis_index("c")
    s = lax.axis_index("s")
    wid = s * 2 + c
    b = wid // 8
    base = (wid % 8) * PW

    def _chunk(t, _):
        o = base + t * CH
        pltpu.sync_copy(sidx_hbm.at[b, pl.ds(o, CH)], idxg)
        for g in range(CH // 16):
            idxg[pl.ds(g * 16, 16)] = idxg[pl.ds(g * 16, 16)] + b * N
        pltpu.sync_copy(sscore_hbm.at[b, pl.ds(o, CH)], sv)
        pltpu.async_copy(xf_hbm.at[idxg], xrows, sem).wait()

        def _scale(r, _):
            bc = plsc.load_gather(sv, [jnp.full((16,), 1, jnp.int32) * r])
            for g in range(F // 16):
                xrows[r, pl.ds(g * 16, 16)] = xrows[r, pl.ds(g * 16, 16)] * bc
            return 0
        lax.fori_loop(0, CH, _scale, 0)
        pltpu.sync_copy(xrows, xo_hbm.at[b, pl.ds(o, CH)])
        return 0
    lax.fori_loop(0, NCH, _chunk, 0)


@functools.partial(
    pl.kernel,
    out_type=[
        jax.ShapeDtypeStruct((B, 8 * PW, 16), jnp.float32),
    ],
    scratch_types=[
        pltpu.VMEM((CH,), jnp.int32),
        pltpu.VMEM((CH, 16), jnp.float32),
        pltpu.SemaphoreType.DMA,
    ],
    compiler_params=pltpu.CompilerParams(needs_layout_passes=False,
                                         use_tc_tiling_on_sc=False),
    mesh=_mesh,
)
def _pgather_kernel(pp_hbm, sidx_hbm, po_hbm, idxg, prows, sem):
    c = lax.axis_index("c")
    s = lax.axis_index("s")
    wid = s * 2 + c
    b = wid // 8
    base = (wid % 8) * PW

    def _chunk(t, _):
        o = base + t * CH
        pltpu.sync_copy(sidx_hbm.at[b, pl.ds(o, CH)], idxg)
        for g in range(CH // 16):
            idxg[pl.ds(g * 16, 16)] = idxg[pl.ds(g * 16, 16)] + b * N
        pltpu.async_copy(pp_hbm.at[idxg], prows, sem).wait()
        pltpu.sync_copy(prows, po_hbm.at[b, pl.ds(o, CH)])
        return 0
    lax.fori_loop(0, NCH, _chunk, 0)


def kernel(x, posi, weight):
    b, n, f = x.shape
    xf = x.reshape(b * n, f)
    score = jnp.tanh((xf * weight).sum(-1)
                     / jnp.sqrt(jnp.sum(weight * weight, -1))).reshape(b, n)
    score_p = jnp.pad(score, ((0, 0), (0, N2 - n)), constant_values=NEG_INF)
    sidx, sscore = _sort_kernel(score_p)
    pp = jnp.pad(posi, ((0, 0), (0, 0), (0, 13))).reshape(b * n, 16)
    xo = _gather_kernel(xf, sidx, sscore)
    xo = xo[0] if isinstance(xo, (list, tuple)) else xo
    po = _pgather_kernel(pp, sidx)
    po = po[0] if isinstance(po, (list, tuple)) else po
    k = n // 2
    return (xo[:, :k], po[:, :k, :3])
